# baseline (speedup 1.0000x reference)
import functools

import jax
import jax.numpy as jnp
from jax import lax
from jax.experimental import pallas as pl
from jax.experimental.pallas import tpu as pltpu
from jax.experimental.pallas import tpu_sc as plsc

N_NODES = 10000
N_EDGES = 320000
D = 128

NC = 2
NS = 16
NW = NC * NS
EPW = N_EDGES // NW
CHUNK = 128
NCHUNK = EPW // CHUNK
TAIL = EPW - NCHUNK * CHUNK
N_PAD = 10112
ROWS_PER_TILE = N_PAD // NS
NBUF = 3

_sc_mesh = plsc.VectorSubcoreMesh(core_axis_name="c", subcore_axis_name="s")


@functools.partial(
    pl.kernel,
    out_type=jax.ShapeDtypeStruct((NC, N_PAD, D), jnp.float32),
    mesh=_sc_mesh,
    scratch_types=[
        pltpu.VMEM((CHUNK,), jnp.int32),
        pltpu.VMEM((CHUNK,), jnp.int32),
        pltpu.VMEM((CHUNK,), jnp.int32),
        pltpu.VMEM((CHUNK,), jnp.int32),
        pltpu.VMEM((CHUNK,), jnp.int32),
        pltpu.VMEM((CHUNK,), jnp.int32),
        pltpu.VMEM((CHUNK, D), jnp.float32),
        pltpu.VMEM((CHUNK, D), jnp.float32),
        pltpu.VMEM((CHUNK, D), jnp.float32),
        pltpu.VMEM_SHARED((N_PAD, D), jnp.float32),
        pltpu.SemaphoreType.DMA,
        pltpu.SemaphoreType.DMA,
        pltpu.SemaphoreType.DMA,
        pltpu.SemaphoreType.DMA,
    ],
)
def _sc_aggregate(x_hbm, ei_hbm, out_hbm,
                  si0, si1, si2, di0, di1, di2, rb0, rb1, rb2, acc,
                  sem_i, sem_d, sem_g, sem_s):
    c = lax.axis_index("c")
    s = lax.axis_index("s")
    w = c * NS + s

    sbufs = (si0, si1, si2)
    dbufs = (di0, di1, di2)
    bufs = (rb0, rb1, rb2)

    def zrow(i, carry):
        for k in range(D // 16):
            rb0[i, pl.ds(k * 16, 16)] = jnp.zeros((16,), jnp.float32)
        return carry

    lax.fori_loop(0, CHUNK, zrow, 0)
    r0 = pl.multiple_of(s * ROWS_PER_TILE, 8)
    for t in range(ROWS_PER_TILE // CHUNK):
        pltpu.sync_copy(rb0, acc.at[pl.ds(r0 + t * CHUNK, CHUNK)])
    rem = ROWS_PER_TILE % CHUNK
    if rem:
        pltpu.sync_copy(
            rb0.at[pl.ds(0, rem)],
            acc.at[pl.ds(r0 + ROWS_PER_TILE - rem, rem)])
    e0 = pl.multiple_of(w * EPW, 8)
    plsc.subcore_barrier()

    def start_src_idx(j, b):
        off = pl.multiple_of(j * CHUNK, 8)
        pltpu.async_copy(ei_hbm.at[pl.ds(e0 + off, CHUNK)], sbufs[b], sem_i)

    def wait_src_idx(b):
        pltpu.make_async_copy(ei_hbm.at[pl.ds(0, CHUNK)], sbufs[b],
                              sem_i).wait()

    def start_dst_idx(j, b):
        off = pl.multiple_of(j * CHUNK, 8)
        pltpu.async_copy(ei_hbm.at[pl.ds(N_EDGES + e0 + off, CHUNK)],
                         dbufs[b], sem_d)

    def wait_dst_idx(b):
        pltpu.make_async_copy(ei_hbm.at[pl.ds(0, CHUNK)], dbufs[b],
                              sem_d).wait()

    def start_gather(j, b):
        del j
        pltpu.async_copy(x_hbm.at[sbufs[b]], bufs[b], sem_g)

    def wait_gather(b):
        pltpu.make_async_copy(x_hbm.at[sbufs[b]], bufs[b], sem_g).wait()

    def start_scatter(j, b):
        del j
        pltpu.async_copy(bufs[b], acc.at[dbufs[b]], sem_s, add=True)

    def wait_scatter(b):
        pltpu.make_async_copy(bufs[b], acc.at[dbufs[b]], sem_s).wait()

    last = NCHUNK - 1

    def emit_round(j, static):
        for b in range(NBUF):
            cchunk = j + b
            if static and cchunk > last:
                continue
            wait_gather(b)
            wait_dst_idx(b)
            start_scatter(cchunk, b)
            if not static or cchunk + NBUF <= last:
                start_src_idx(cchunk + NBUF, b)
        for b in range(NBUF):
            if static and j + b > last:
                continue
            wait_scatter(b)
            if not static or j + b + NBUF <= last:
                start_dst_idx(j + b + NBUF, b)
                wait_src_idx(b)
                start_gather(j + b + NBUF, b)

    for b in range(NBUF):
        start_src_idx(b, b)
        start_dst_idx(b, b)
    for b in range(NBUF):
        wait_src_idx(b)
        start_gather(b, b)

    def body(i, carry):
        emit_round(3 * i, False)
        return carry

    lax.fori_loop(0, 25, body, 0)
    emit_round(75, True)
    toff = pl.multiple_of(NCHUNK * CHUNK, 8)
    pltpu.async_copy(ei_hbm.at[pl.ds(e0 + toff, TAIL)],
                     si0.at[pl.ds(0, TAIL)], sem_i)
    pltpu.async_copy(ei_hbm.at[pl.ds(N_EDGES + e0 + toff, TAIL)],
                     di0.at[pl.ds(0, TAIL)], sem_i)
    pltpu.make_async_copy(ei_hbm.at[pl.ds(0, TAIL)],
                          si0.at[pl.ds(0, TAIL)], sem_i).wait()
    pltpu.make_async_copy(ei_hbm.at[pl.ds(0, TAIL)],
                          di0.at[pl.ds(0, TAIL)], sem_i).wait()
    pltpu.async_copy(x_hbm.at[si0.at[pl.ds(0, TAIL)]],
                     rb0.at[pl.ds(0, TAIL)], sem_g)
    pltpu.make_async_copy(x_hbm.at[si0.at[pl.ds(0, TAIL)]],
                          rb0.at[pl.ds(0, TAIL)], sem_g).wait()
    pltpu.sync_copy(rb0.at[pl.ds(0, TAIL)],
                    acc.at[di0.at[pl.ds(0, TAIL)]], add=True)
    plsc.subcore_barrier()

    pltpu.sync_copy(acc.at[pl.ds(r0, ROWS_PER_TILE)],
                    out_hbm.at[c].at[pl.ds(r0, ROWS_PER_TILE)])


_ROW_BLK = 1000
_DN_T = (((1,), (1,)), ((), ()))


def _tc_root(x_ref, wr_ref, b_ref, o_ref):
    o_ref[...] = lax.dot_general(
        x_ref[...], wr_ref[...], _DN_T,
        preferred_element_type=jnp.float32) + b_ref[...]


_tc_root_call = pl.pallas_call(
    _tc_root,
    grid=(N_NODES // _ROW_BLK,),
    in_specs=[
        pl.BlockSpec((_ROW_BLK, D), lambda i: (i, 0)),
        pl.BlockSpec((D, D), lambda i: (0, 0)),
        pl.BlockSpec((1, D), lambda i: (0, 0)),
    ],
    out_specs=pl.BlockSpec((_ROW_BLK, D), lambda i: (i, 0)),
    out_shape=jax.ShapeDtypeStruct((N_NODES, D), jnp.float32),
)


def _tc_fin(p_ref, r_ref, wl_ref, o_ref):
    agg = p_ref[0] + p_ref[1]
    y = lax.dot_general(agg, wl_ref[...], _DN_T,
                        preferred_element_type=jnp.float32)
    o_ref[...] = jnp.tanh(y + r_ref[...])


_tc_fin_call = pl.pallas_call(
    _tc_fin,
    grid=(N_NODES // _ROW_BLK,),
    in_specs=[
        pl.BlockSpec((NC, _ROW_BLK, D), lambda i: (0, i, 0)),
        pl.BlockSpec((_ROW_BLK, D), lambda i: (i, 0)),
        pl.BlockSpec((D, D), lambda i: (0, 0)),
    ],
    out_specs=pl.BlockSpec((_ROW_BLK, D), lambda i: (i, 0)),
    out_shape=jax.ShapeDtypeStruct((N_NODES, D), jnp.float32),
)


def kernel(x, edge_index, W_l, b_l, W_r):
    ei = edge_index.astype(jnp.int32).reshape(-1)
    root = _tc_root_call(x, W_r, b_l.reshape(1, D))
    partials = _sc_aggregate(x, ei)
    return _tc_fin_call(partials, root, W_l)
